# baseline (speedup 1.0000x reference)
import jax
import jax.numpy as jnp
from jax.experimental import pallas as pl
from jax.experimental.pallas import tpu as pltpu

_N = 10000
_TM1 = 80
_TM = 400
_NBLK = _N // _TM
_F32 = jnp.float32
_BF16 = jnp.bfloat16
_PARAMS = pltpu.CompilerParams(vmem_limit_bytes=120 * 1024 * 1024)

_STREAM_SPEC = pl.BlockSpec(
    (_TM, _N), lambda i: (i, 0), pipeline_mode=pl.Buffered(buffer_count=4)
)


def _lrelu(y):
    return jnp.where(y > 0, y, 0.01 * y)


def _sup1_body(x_ref, w_ref, o_ref):
    o_ref[...] = jnp.dot(
        x_ref[...], w_ref[...], preferred_element_type=_F32
    ).astype(_BF16)


def _layer1_body(adj_ref, s_ref, w_ref, b_ref, a16_hbm_ref, sup_ref, cnt_ref):
    cnt_ref[0] = 0

    def inner(a_ref, a16_ref):
        i = cnt_ref[0]
        cnt_ref[0] = i + 1
        a16 = a_ref[...].astype(_BF16)
        a16_ref[...] = a16
        y = jnp.dot(a16, s_ref[...], preferred_element_type=_F32) + b_ref[...]
        h = _lrelu(y)
        sup_ref[pl.ds(i * _TM1, _TM1), :] = jnp.dot(
            h, w_ref[...], preferred_element_type=_F32
        ).astype(_BF16)

    pltpu.emit_pipeline(
        inner,
        grid=(_N // _TM1,),
        in_specs=[pl.BlockSpec((_TM1, _N), lambda i: (i, 0),
                               pipeline_mode=pl.Buffered(buffer_count=6))],
        out_specs=[pl.BlockSpec((_TM1, _N), lambda i: (i, 0))],
    )(adj_ref, a16_hbm_ref)


def _deep_body(adj_ref, s2_ref, w3_ref, w4_ref, w5_ref, w6_ref, w7_ref, w8_ref,
               b2_ref, b3_ref, b4_ref, b5_ref, b6_ref, b7_ref, b8_ref,
               lat_ref, out_ref, supa_ref, supb_ref, cnt_ref):

    def run_layer(step):
        cnt_ref[0] = 0

        def inner(a_ref):
            i = cnt_ref[0]
            cnt_ref[0] = i + 1
            step(a_ref[...], pl.ds(i * _TM, _TM))

        pltpu.emit_pipeline(
            inner, grid=(_NBLK,), in_specs=[_STREAM_SPEC]
        )(adj_ref)

    def l2(a, rows):
        h = _lrelu(jnp.dot(a, s2_ref[...], preferred_element_type=_F32) + b2_ref[...])
        supa_ref[rows, :16] = jnp.dot(h, w3_ref[...], preferred_element_type=_F32).astype(_BF16)

    def l3(a, rows):
        h = _lrelu(jnp.dot(a, supa_ref[:, :16], preferred_element_type=_F32) + b3_ref[...])
        supb_ref[rows, :8] = jnp.dot(h, w4_ref[...], preferred_element_type=_F32).astype(_BF16)

    def l4(a, rows):
        y = jnp.dot(a, supb_ref[:, :8], preferred_element_type=_F32) + b4_ref[...]
        lat_ref[rows, :] = y
        supa_ref[rows, :16] = jnp.dot(y, w5_ref[...], preferred_element_type=_F32).astype(_BF16)

    def l5(a, rows):
        h = _lrelu(jnp.dot(a, supa_ref[:, :16], preferred_element_type=_F32) + b5_ref[...])
        supb_ref[rows, :32] = jnp.dot(h, w6_ref[...], preferred_element_type=_F32).astype(_BF16)

    def l6(a, rows):
        h = _lrelu(jnp.dot(a, supb_ref[:, :32], preferred_element_type=_F32) + b6_ref[...])
        supa_ref[rows, :64] = jnp.dot(h, w7_ref[...], preferred_element_type=_F32).astype(_BF16)

    def l7(a, rows):
        h = _lrelu(jnp.dot(a, supa_ref[:, :64], preferred_element_type=_F32) + b7_ref[...])
        supb_ref[rows, :] = jnp.dot(h, w8_ref[...], preferred_element_type=_F32).astype(_BF16)

    def l8(a, rows):
        out_ref[rows, :] = jnp.dot(a, supb_ref[...], preferred_element_type=_F32) + b8_ref[...]

    for step in (l2, l3, l4, l5, l6, l7, l8):
        run_layer(step)


def _row_spec(tm, d):
    return pl.BlockSpec((tm, d), lambda i: (i, 0))


def _full_spec(r, c):
    return pl.BlockSpec((r, c), lambda i: (0, 0))


def kernel(x, adj, inv_adj, W1, b1, W2, b2, W3, b3, W4, b4, W5, b5, W6, b6,
           W7, b7, W8, b8):
    del inv_adj
    n, d0 = x.shape
    bs = [b.reshape(1, -1) for b in (b1, b2, b3, b4, b5, b6, b7, b8)]

    sup1 = pl.pallas_call(
        _sup1_body,
        grid=(pl.cdiv(n, 800),),
        in_specs=[_row_spec(800, d0), _full_spec(d0, 64)],
        out_specs=_row_spec(800, 64),
        out_shape=jax.ShapeDtypeStruct((n, 64), _BF16),
        compiler_params=_PARAMS,
    )(x, W1)

    vmem = pl.BlockSpec(memory_space=pltpu.VMEM)
    adj16, sup2 = pl.pallas_call(
        _layer1_body,
        in_specs=[pl.BlockSpec(memory_space=pl.ANY), vmem, vmem, vmem],
        out_specs=[pl.BlockSpec(memory_space=pl.ANY), vmem],
        out_shape=[
            jax.ShapeDtypeStruct((n, n), _BF16),
            jax.ShapeDtypeStruct((n, 32), _BF16),
        ],
        scratch_shapes=[pltpu.SMEM((1,), jnp.int32)],
        compiler_params=_PARAMS,
    )(adj, sup1, W2, bs[0])

    lat, out = pl.pallas_call(
        _deep_body,
        in_specs=[pl.BlockSpec(memory_space=pl.ANY)] + [vmem] * 14,
        out_specs=[vmem, vmem],
        out_shape=[
            jax.ShapeDtypeStruct((n, 8), _F32),
            jax.ShapeDtypeStruct((n, 128), _F32),
        ],
        scratch_shapes=[
            pltpu.VMEM((n, 64), _BF16),
            pltpu.VMEM((n, 128), _BF16),
            pltpu.SMEM((1,), jnp.int32),
        ],
        compiler_params=_PARAMS,
    )(adj16, sup2, W3, W4, W5, W6, W7, W8, *bs[1:])

    return (lat, out)
